# baseline (speedup 1.0000x reference)
import functools

import jax
import jax.numpy as jnp
from jax import lax
from jax.experimental import pallas as pl
from jax.experimental.pallas import tpu as pltpu
from jax.experimental.pallas import tpu_sc as plsc

VOCAB = 50257
N_POS = 2048
N_EMBD = 768
BATCH = 4
SEQ = 2048

NW = 32
PPW = SEQ // NW
SUB = 32
NSUB = BATCH * PPW // SUB
NBUF = 3
LANES = 16
VPR = N_EMBD // LANES

_mesh = plsc.VectorSubcoreMesh(core_axis_name="c", subcore_axis_name="s")


@functools.partial(
    pl.kernel,
    mesh=_mesh,
    out_type=jax.ShapeDtypeStruct((BATCH, SEQ, N_EMBD), jnp.float32),
    scratch_types=[pltpu.VMEM((PPW, N_EMBD), jnp.float32)]
    + [pltpu.VMEM((SUB,), jnp.int32) for _ in range(NSUB)]
    + [pltpu.VMEM((SUB, N_EMBD), jnp.float32) for _ in range(NBUF)]
    + [pltpu.SemaphoreType.DMA for _ in range(1 + 2 * NBUF)],
)
def _embd_sc(ids_hbm, wte_hbm, wpe_hbm, out_hbm, wpe_v,
             i0, i1, i2, i3, i4, i5, i6, i7,
             buf0, buf1, buf2, sem_wpe, sg0, sg1, sg2, ss0, ss1, ss2):
    idxs = (i0, i1, i2, i3, i4, i5, i6, i7)
    bufs = (buf0, buf1, buf2)
    sg = (sg0, sg1, sg2)
    ss = (ss0, ss1, ss2)
    wid = lax.axis_index("s") * 2 + lax.axis_index("c")
    pos0 = wid * PPW

    wpe_dma = pltpu.async_copy(wpe_hbm.at[pl.ds(pos0, PPW)], wpe_v, sem_wpe)

    def batch_of(j):
        return j // 2

    def poff_of(j):
        return (j % 2) * SUB

    def load_idx(j):
        pltpu.sync_copy(
            ids_hbm.at[batch_of(j), pl.ds(pos0 + poff_of(j), SUB)], idxs[j])

    def start_gather(j):
        k = j % NBUF
        return pltpu.async_copy(wte_hbm.at[idxs[j]], bufs[k], sg[k])

    load_idx(0)
    load_idx(1)
    gathers = [None] * NSUB
    stores = [None] * NSUB
    gathers[0] = start_gather(0)
    gathers[1] = start_gather(1)
    for j in range(2, NSUB):
        load_idx(j)
    wpe_dma.wait()

    for j in range(NSUB):
        k = j % NBUF
        gathers[j].wait()
        prow = poff_of(j)
        buf = bufs[k]

        def row_body(r, carry, buf=buf, prow=prow):
            for i in range(VPR):
                sl = pl.ds(i * LANES, LANES)
                plsc.addupdate(buf.at[r, sl], wpe_v[prow + r, sl])
            return carry

        lax.fori_loop(0, SUB, row_body, 0)
        stores[j] = pltpu.async_copy(
            buf, out_hbm.at[batch_of(j), pl.ds(pos0 + poff_of(j), SUB)],
            ss[k])
        if j + 2 < NSUB:
            if stores[j + 2 - NBUF] is not None:
                stores[j + 2 - NBUF].wait()
            gathers[j + 2] = start_gather(j + 2)

    for j in range(NSUB - NBUF, NSUB):
        stores[j].wait()


@jax.jit
def kernel(input_ids, wte, wpe):
    return _embd_sc(input_ids.astype(jnp.int32), wte, wpe)


# speedup vs baseline: 1.0622x; 1.0622x over previous
import functools

import jax
import jax.numpy as jnp
from jax import lax
from jax.experimental import pallas as pl
from jax.experimental.pallas import tpu as pltpu
from jax.experimental.pallas import tpu_sc as plsc

VOCAB = 50257
N_POS = 2048
N_EMBD = 768
BATCH = 4
SEQ = 2048

NW = 32
NTILE = 16
PPW = SEQ // NW
SUB = 32
NSUB = BATCH * PPW // SUB
NBUF = 2
LANES = 16
VPR = N_EMBD // LANES

_mesh = plsc.VectorSubcoreMesh(core_axis_name="c", subcore_axis_name="s")


@functools.partial(
    pl.kernel,
    mesh=_mesh,
    out_type=jax.ShapeDtypeStruct((BATCH, SEQ, N_EMBD), jnp.float32),
    scratch_types=[pltpu.VMEM((SUB, N_EMBD), jnp.float32)]
    + [pltpu.VMEM((SUB,), jnp.int32) for _ in range(NSUB)]
    + [pltpu.VMEM((SUB, N_EMBD), jnp.float32) for _ in range(NBUF)]
    + [pltpu.VMEM_SHARED((NBUF, NTILE, SUB, N_EMBD), jnp.float32)]
    + [pltpu.SemaphoreType.DMA for _ in range(1 + 3 * NBUF)],
)
def _embd_sc(ids_hbm, wte_hbm, wpe_hbm, out_hbm, wpe_v,
             i0, i1, i2, i3, i4, i5, i6, i7,
             buf0, buf1, spm, sem_wpe, sg0, sg1, sx0, sx1, sd0, sd1):
    idxs = (i0, i1, i2, i3, i4, i5, i6, i7)
    bufs = (buf0, buf1)
    sg = (sg0, sg1)
    sx = (sx0, sx1)
    sd = (sd0, sd1)
    sid = lax.axis_index("s")
    wid = sid * 2 + lax.axis_index("c")
    pos0 = wid * PPW

    def batch_of(j):
        return j % 4

    def poff_of(j):
        return (j // 4) * SUB

    def load_idx(j):
        pltpu.sync_copy(
            ids_hbm.at[batch_of(j), pl.ds(pos0 + poff_of(j), SUB)], idxs[j])

    def start_gather(j):
        k = j % NBUF
        return pltpu.async_copy(wte_hbm.at[idxs[j]], bufs[k], sg[k])

    wpe_dma = pltpu.async_copy(wpe_hbm.at[pl.ds(pos0, SUB)], wpe_v, sem_wpe)
    load_idx(0)
    load_idx(1)
    gathers = [None] * NSUB
    stores = [None] * NSUB
    gathers[0] = start_gather(0)
    gathers[1] = start_gather(1)
    for j in range(2, NSUB):
        load_idx(j)

    for j in range(NSUB):
        k = j % NBUF
        gathers[j].wait()
        if j == 0 or j == 4:
            wpe_dma.wait()
        buf = bufs[k]

        def row_body(r, carry, buf=buf):
            for i in range(VPR):
                sl = pl.ds(i * LANES, LANES)
                plsc.addupdate(buf.at[r, sl], wpe_v[r, sl])
            return carry

        lax.fori_loop(0, SUB, row_body, 0)
        if j == 3:
            wpe_dma = pltpu.async_copy(
                wpe_hbm.at[pl.ds(pos0 + SUB, SUB)], wpe_v, sem_wpe)
        if stores[j - NBUF] is not None:
            stores[j - NBUF].wait()
        pltpu.async_copy(buf, spm.at[k, sid], sx[k]).wait()
        stores[j] = pltpu.async_copy(
            spm.at[k, sid],
            out_hbm.at[batch_of(j), pl.ds(pos0 + poff_of(j), SUB)], sd[k])
        if j + 2 < NSUB:
            gathers[j + 2] = start_gather(j + 2)

    for j in range(NSUB - NBUF, NSUB):
        stores[j].wait()


@jax.jit
def kernel(input_ids, wte, wpe):
    return _embd_sc(input_ids.astype(jnp.int32), wte, wpe)


# speedup vs baseline: 1.1520x; 1.0845x over previous
import functools

import jax
import jax.numpy as jnp
from jax import lax
from jax.experimental import pallas as pl
from jax.experimental.pallas import tpu as pltpu
from jax.experimental.pallas import tpu_sc as plsc

VOCAB = 50257
N_POS = 2048
N_EMBD = 768
BATCH = 4
SEQ = 2048

NW = 32
PPW = SEQ // NW
SUB = 32
NSUB = BATCH * PPW // SUB
NBUF = 4
LANES = 16
VPR = N_EMBD // LANES

_mesh = plsc.VectorSubcoreMesh(core_axis_name="c", subcore_axis_name="s")


@functools.partial(
    pl.kernel,
    mesh=_mesh,
    out_type=jax.ShapeDtypeStruct((BATCH, SEQ, N_EMBD), jnp.float32),
    scratch_types=[pltpu.VMEM((SUB, N_EMBD), jnp.float32)]
    + [pltpu.VMEM((SUB,), jnp.int32) for _ in range(NSUB)]
    + [pltpu.VMEM((SUB, N_EMBD), jnp.float32) for _ in range(NBUF)]
    + [pltpu.SemaphoreType.DMA for _ in range(1 + 2 * NBUF)],
)
def _embd_sc(ids_hbm, wte_hbm, wpe_hbm, out_hbm, wpe_v,
             i0, i1, i2, i3, i4, i5, i6, i7,
             buf0, buf1, buf2, buf3, sem_wpe,
             sg0, sg1, sg2, sg3, ss0, ss1, ss2, ss3):
    idxs = (i0, i1, i2, i3, i4, i5, i6, i7)
    bufs = (buf0, buf1, buf2, buf3)
    sg = (sg0, sg1, sg2, sg3)
    ss = (ss0, ss1, ss2, ss3)
    wid = lax.axis_index("s") * 2 + lax.axis_index("c")
    pos0 = wid * PPW

    def batch_of(j):
        return j % 4

    def poff_of(j):
        return (j // 4) * SUB

    def load_idx(j):
        pltpu.sync_copy(
            ids_hbm.at[batch_of(j), pl.ds(pos0 + poff_of(j), SUB)], idxs[j])

    def start_gather(j):
        k = j % NBUF
        return pltpu.async_copy(wte_hbm.at[idxs[j]], bufs[k], sg[k])

    wpe_dma = pltpu.async_copy(wpe_hbm.at[pl.ds(pos0, SUB)], wpe_v, sem_wpe)
    for j in range(NBUF - 1):
        load_idx(j)
    gathers = [None] * NSUB
    stores = [None] * NSUB
    for j in range(NBUF - 1):
        gathers[j] = start_gather(j)
    for j in range(NBUF - 1, NSUB):
        load_idx(j)

    for j in range(NSUB):
        k = j % NBUF
        gathers[j].wait()
        if j == 0 or j == 4:
            wpe_dma.wait()
        buf = bufs[k]

        def row_body(r, carry, buf=buf):
            for i in range(VPR):
                sl = pl.ds(i * LANES, LANES)
                plsc.addupdate(buf.at[r, sl], wpe_v[r, sl])
            return carry

        lax.fori_loop(0, SUB, row_body, 0)
        if j == 3:
            wpe_dma = pltpu.async_copy(
                wpe_hbm.at[pl.ds(pos0 + SUB, SUB)], wpe_v, sem_wpe)
        stores[j] = pltpu.async_copy(
            buf, out_hbm.at[batch_of(j), pl.ds(pos0 + poff_of(j), SUB)],
            ss[k])
        if j + NBUF - 1 < NSUB:
            if j >= 1:
                stores[j - 1].wait()
            gathers[j + NBUF - 1] = start_gather(j + NBUF - 1)

    for j in range(NSUB - NBUF, NSUB):
        stores[j].wait()


@jax.jit
def kernel(input_ids, wte, wpe):
    return _embd_sc(input_ids.astype(jnp.int32), wte, wpe)
